# baseline (speedup 1.0000x reference)
import functools

import jax
import jax.numpy as jnp
from jax import lax
from jax.experimental import pallas as pl
from jax.experimental.pallas import tpu as pltpu
from jax.experimental.pallas import tpu_sc as plsc

_H = 768
_E = 16
_TOPK = 2
_CAP = 256
_T = 2048
_TB = 512
_G = _T // _TB
_EC = _E * _CAP

_NW = 32
_TPW = _T // _NW
_ROWS = 8
_NCH = _TPW // _ROWS
_EPW = _TPW * _TOPK


def _routing_body(x_ref, w1_ref, b1_ref, w2_ref, b2_ref, ew_ref,
                  q_ref, wgt_ref, probs_ref, aux_ref, acc_ref):
    i = pl.program_id(0)

    @pl.when(i == 0)
    def _init():
        acc_ref[...] = jnp.zeros_like(acc_ref)

    h = jnp.dot(x_ref[...], w1_ref[...], preferred_element_type=jnp.float32)
    h = jnp.maximum(h + b1_ref[...], 0.0)
    logits = jnp.dot(h, w2_ref[...], preferred_element_type=jnp.float32)
    logits = (logits + b2_ref[...]) * ew_ref[...]

    m = jnp.max(logits, axis=1, keepdims=True)
    ex = jnp.exp(logits - m)
    p = ex / jnp.sum(ex, axis=1, keepdims=True)
    probs_ref[...] = p

    iota_e = lax.broadcasted_iota(jnp.int32, (_TB, _E), 1)
    p0 = jnp.max(p, axis=1, keepdims=True)
    i0 = jnp.min(jnp.where(p == p0, iota_e, _E), axis=1, keepdims=True)
    oh0 = iota_e == i0
    pm = jnp.where(oh0, -1.0, p)
    p1 = jnp.max(pm, axis=1, keepdims=True)
    i1 = jnp.min(jnp.where(pm == p1, iota_e, _E), axis=1, keepdims=True)
    oh1 = iota_e == i1
    s = p0 + p1 + 1e-8
    w0 = p0 / s
    w1 = p1 / s

    oh0f = oh0.astype(jnp.float32)
    oh1f = oh1.astype(jnp.float32)
    s2 = oh0f + oh1f
    row = lax.broadcasted_iota(jnp.int32, (_TB, _TB), 0)
    col = lax.broadcasted_iota(jnp.int32, (_TB, _TB), 1)
    stril = (row > col).astype(jnp.float32)
    c_in = jnp.dot(stril, s2, preferred_element_type=jnp.float32)
    carry = acc_ref[0:1, 0:_E]
    c_tot = c_in + carry
    pos0 = jnp.sum(c_tot * oh0f, axis=1, keepdims=True).astype(jnp.int32)
    pos1 = jnp.sum(c_tot * oh1f, axis=1, keepdims=True).astype(jnp.int32)
    acc_ref[0:1, 0:_E] = carry + jnp.sum(s2, axis=0, keepdims=True)
    acc_ref[1:2, 0:_E] += jnp.sum(p, axis=0, keepdims=True)

    q0 = jnp.where(pos0 < _CAP, i0 * _CAP + pos0, -1)
    q1 = jnp.where(pos1 < _CAP, i1 * _CAP + pos1, -1)
    q_ref[...] = jnp.concatenate([q0, q1], axis=1)
    wgt_ref[...] = jnp.concatenate([w0, w1], axis=1)

    cnt = acc_ref[0:1, 0:_E]
    psum = acc_ref[1:2, 0:_E]
    aux_ref[...] = (0.1 * _E) * jnp.sum(
        (psum / _T) * (cnt / (_T * _TOPK)), axis=1, keepdims=True)


def _tc_routing(x, w1t, b1r, w2t, b2r, ewr):
    return pl.pallas_call(
        _routing_body,
        grid=(_G,),
        in_specs=[
            pl.BlockSpec((_TB, _H), lambda i: (i, 0)),
            pl.BlockSpec((_H, _H), lambda i: (0, 0)),
            pl.BlockSpec((1, _H), lambda i: (0, 0)),
            pl.BlockSpec((_H, _E), lambda i: (0, 0)),
            pl.BlockSpec((1, _E), lambda i: (0, 0)),
            pl.BlockSpec((1, _E), lambda i: (0, 0)),
        ],
        out_specs=[
            pl.BlockSpec((_TB, 2), lambda i: (i, 0)),
            pl.BlockSpec((_TB, 2), lambda i: (i, 0)),
            pl.BlockSpec((_TB, _E), lambda i: (i, 0)),
            pl.BlockSpec((1, 1), lambda i: (0, 0)),
        ],
        out_shape=[
            jax.ShapeDtypeStruct((_T, 2), jnp.int32),
            jax.ShapeDtypeStruct((_T, 2), jnp.float32),
            jax.ShapeDtypeStruct((_T, _E), jnp.float32),
            jax.ShapeDtypeStruct((1, 1), jnp.float32),
        ],
        scratch_shapes=[pltpu.VMEM((8, 128), jnp.float32)],
    )(x, w1t, b1r, w2t, b2r, ewr)


_CHW = _ROWS * _EC


def _sc_fill_body(q_hbm, w_hbm, disp_hbm, comb_hbm, qv, wv, dbuf, cbuf, sem):
    wid = lax.axis_index("s") * 2 + lax.axis_index("c")
    ebase = wid * _EPW

    pltpu.sync_copy(q_hbm.at[pl.ds(ebase, _EPW)], qv)
    pltpu.sync_copy(w_hbm.at[pl.ds(ebase, _EPW)], wv)

    def _zero(j, _):
        e = j // 16
        col = (j % 16) * 16
        for r in range(_ROWS):
            dbuf[r, e, pl.ds(col, 16)] = jnp.zeros((16,), jnp.float32)
            cbuf[r, e, pl.ds(col, 16)] = jnp.zeros((16,), jnp.float32)
        return 0

    lax.fori_loop(0, _E * 16, _zero, 0)

    lanes = lax.iota(jnp.int32, 16)
    tok_pat = lax.shift_right_logical(lanes, 1)
    ones = jnp.full((16,), 1.0, jnp.float32)
    zeros = jnp.zeros((16,), jnp.float32)

    def _chunk(c, _):
        q = qv[pl.ds(c * 16, 16)]
        w = wv[pl.ds(c * 16, 16)]
        msk = q >= 0
        qs = jnp.where(msk, q, 0)
        ei = lax.shift_right_logical(qs, 8)
        ci = qs & 255
        plsc.store_scatter(dbuf, [tok_pat, ei, ci], ones, mask=msk)
        plsc.store_scatter(cbuf, [tok_pat, ei, ci], w, mask=msk)
        rowb = wid * _TPW + c * _ROWS
        cp_d = pltpu.async_copy(dbuf, disp_hbm.at[pl.ds(rowb, _ROWS)], sem)
        cp_c = pltpu.async_copy(cbuf, comb_hbm.at[pl.ds(rowb, _ROWS)], sem)
        cp_d.wait()
        cp_c.wait()
        plsc.store_scatter(dbuf, [tok_pat, ei, ci], zeros, mask=msk)
        plsc.store_scatter(cbuf, [tok_pat, ei, ci], zeros, mask=msk)
        return 0

    lax.fori_loop(0, _NCH, _chunk, 0)


_sc_fill = functools.partial(
    pl.kernel,
    out_type=[
        jax.ShapeDtypeStruct((_T, _E, _CAP), jnp.float32),
        jax.ShapeDtypeStruct((_T, _E, _CAP), jnp.float32),
    ],
    mesh=plsc.VectorSubcoreMesh(core_axis_name="c", subcore_axis_name="s"),
    scratch_types=[
        pltpu.VMEM((_EPW,), jnp.int32),
        pltpu.VMEM((_EPW,), jnp.float32),
        pltpu.VMEM((_ROWS, _E, _CAP), jnp.float32),
        pltpu.VMEM((_ROWS, _E, _CAP), jnp.float32),
        pltpu.SemaphoreType.DMA,
    ],
    compiler_params=pltpu.CompilerParams(
        needs_layout_passes=False, use_tc_tiling_on_sc=True),
)(_sc_fill_body)


def kernel(hidden_states, W1, b1, W2, b2, expert_weights):
    Bv, Sv, Hv = hidden_states.shape
    x = hidden_states.reshape(Bv * Sv, Hv)
    q2, wgt2, probs, aux = _tc_routing(
        x, W1.T, b1.reshape(1, Hv), W2.T, b2.reshape(1, _E),
        expert_weights.reshape(1, _E))
    q = q2.reshape(_T * _TOPK)
    wgt = wgt2.reshape(_T * _TOPK)
    disp, comb = _sc_fill(q, wgt)
    dispatch = disp.reshape(Bv, Sv, _E, _CAP)
    combine = comb.reshape(Bv, Sv, _E, _CAP)
    router_probs = probs.reshape(Bv, Sv, _E)
    return dispatch, combine, router_probs, aux.reshape(())


# speedup vs baseline: 1.5210x; 1.5210x over previous
import jax
import jax.numpy as jnp
from jax import lax
from jax.experimental import pallas as pl
from jax.experimental.pallas import tpu as pltpu

_H = 768
_E = 16
_TOPK = 2
_CAP = 256
_T = 2048
_TB = 256
_G = _T // _TB


def _router_body(x_ref, w1_ref, b1_ref, w2_ref, b2_ref, ew_ref,
                 disp_ref, comb_ref, probs_ref, aux_ref, acc_ref):
    i = pl.program_id(0)

    @pl.when(i == 0)
    def _init():
        acc_ref[...] = jnp.zeros_like(acc_ref)

    h = jnp.dot(x_ref[...], w1_ref[...], preferred_element_type=jnp.float32)
    h = jnp.maximum(h + b1_ref[...], 0.0)
    logits = jnp.dot(h, w2_ref[...], preferred_element_type=jnp.float32)
    logits = (logits + b2_ref[...]) * ew_ref[...]

    m = jnp.max(logits, axis=1, keepdims=True)
    ex = jnp.exp(logits - m)
    p = ex / jnp.sum(ex, axis=1, keepdims=True)
    probs_ref[...] = p

    iota_e = lax.broadcasted_iota(jnp.int32, (_TB, _E), 1)
    p0 = jnp.max(p, axis=1, keepdims=True)
    i0 = jnp.min(jnp.where(p == p0, iota_e, _E), axis=1, keepdims=True)
    oh0 = iota_e == i0
    pm = jnp.where(oh0, -1.0, p)
    p1 = jnp.max(pm, axis=1, keepdims=True)
    i1 = jnp.min(jnp.where(pm == p1, iota_e, _E), axis=1, keepdims=True)
    oh1 = iota_e == i1
    s = p0 + p1 + 1e-8
    w0 = p0 / s
    w1 = p1 / s

    oh0f = oh0.astype(jnp.float32)
    oh1f = oh1.astype(jnp.float32)
    s2 = oh0f + oh1f
    row = lax.broadcasted_iota(jnp.int32, (_TB, _TB), 0)
    col = lax.broadcasted_iota(jnp.int32, (_TB, _TB), 1)
    stril = (row > col).astype(jnp.float32)
    c_in = jnp.dot(stril, s2, preferred_element_type=jnp.float32)
    carry = acc_ref[0:1, 0:_E]
    c_tot = c_in + carry
    pos0 = jnp.sum(c_tot * oh0f, axis=1, keepdims=True).astype(jnp.int32)
    pos1 = jnp.sum(c_tot * oh1f, axis=1, keepdims=True).astype(jnp.int32)
    acc_ref[0:1, 0:_E] = carry + jnp.sum(s2, axis=0, keepdims=True)
    acc_ref[1:2, 0:_E] += jnp.sum(p, axis=0, keepdims=True)

    q0 = jnp.where(pos0 < _CAP, i0 * _CAP + pos0, -1)
    q1 = jnp.where(pos1 < _CAP, i1 * _CAP + pos1, -1)
    q0_3 = jnp.expand_dims(q0, 2)
    q1_3 = jnp.expand_dims(q1, 2)
    w0_3 = jnp.expand_dims(w0, 2)
    w1_3 = jnp.expand_dims(w1, 2)
    ee = lax.broadcasted_iota(jnp.int32, (_TB, _E, _CAP), 1)
    cc = lax.broadcasted_iota(jnp.int32, (_TB, _E, _CAP), 2)
    qq = ee * _CAP + cc
    m0 = qq == q0_3
    m1 = qq == q1_3
    disp_ref[...] = m0.astype(jnp.float32) + m1.astype(jnp.float32)
    comb_ref[...] = jnp.where(m0, w0_3, 0.0) + jnp.where(m1, w1_3, 0.0)

    cnt = acc_ref[0:1, 0:_E]
    psum = acc_ref[1:2, 0:_E]
    aux_ref[...] = (0.1 * _E) * jnp.sum(
        (psum / _T) * (cnt / (_T * _TOPK)), axis=1, keepdims=True)


def kernel(hidden_states, W1, b1, W2, b2, expert_weights):
    Bv, Sv, Hv = hidden_states.shape
    x = hidden_states.reshape(Bv * Sv, Hv)

    disp, comb, probs, aux = pl.pallas_call(
        _router_body,
        grid=(_G,),
        in_specs=[
            pl.BlockSpec((_TB, _H), lambda i: (i, 0)),
            pl.BlockSpec((_H, _H), lambda i: (0, 0)),
            pl.BlockSpec((1, _H), lambda i: (0, 0)),
            pl.BlockSpec((_H, _E), lambda i: (0, 0)),
            pl.BlockSpec((1, _E), lambda i: (0, 0)),
            pl.BlockSpec((1, _E), lambda i: (0, 0)),
        ],
        out_specs=[
            pl.BlockSpec((_TB, _E, _CAP), lambda i: (i, 0, 0)),
            pl.BlockSpec((_TB, _E, _CAP), lambda i: (i, 0, 0)),
            pl.BlockSpec((_TB, _E), lambda i: (i, 0)),
            pl.BlockSpec((1, 1), lambda i: (0, 0)),
        ],
        out_shape=[
            jax.ShapeDtypeStruct((_T, _E, _CAP), jnp.float32),
            jax.ShapeDtypeStruct((_T, _E, _CAP), jnp.float32),
            jax.ShapeDtypeStruct((_T, _E), jnp.float32),
            jax.ShapeDtypeStruct((1, 1), jnp.float32),
        ],
        scratch_shapes=[pltpu.VMEM((8, 128), jnp.float32)],
    )(x, W1.T, b1.reshape(1, Hv), W2.T, b2.reshape(1, _E),
      expert_weights.reshape(1, _E))

    dispatch = disp.reshape(Bv, Sv, _E, _CAP)
    combine = comb.reshape(Bv, Sv, _E, _CAP)
    router_probs = probs.reshape(Bv, Sv, _E)
    return dispatch, combine, router_probs, aux.reshape(())


# speedup vs baseline: 1.5508x; 1.0196x over previous
import jax
import jax.numpy as jnp
from jax import lax
from jax.experimental import pallas as pl
from jax.experimental.pallas import tpu as pltpu

_H = 768
_E = 16
_TOPK = 2
_CAP = 256
_T = 2048
_TB = 512
_G = _T // _TB


def _router_body(x_ref, w1_ref, b1_ref, w2_ref, b2_ref, ew_ref,
                 disp_ref, comb_ref, probs_ref, aux_ref, acc_ref):
    i = pl.program_id(0)

    @pl.when(i == 0)
    def _init():
        acc_ref[...] = jnp.zeros_like(acc_ref)

    h = jnp.dot(x_ref[...], w1_ref[...], preferred_element_type=jnp.float32)
    h = jnp.maximum(h + b1_ref[...], 0.0)
    logits = jnp.dot(h, w2_ref[...], preferred_element_type=jnp.float32)
    logits = (logits + b2_ref[...]) * ew_ref[...]

    m = jnp.max(logits, axis=1, keepdims=True)
    ex = jnp.exp(logits - m)
    p = ex / jnp.sum(ex, axis=1, keepdims=True)
    probs_ref[...] = p

    iota_e = lax.broadcasted_iota(jnp.int32, (_TB, _E), 1)
    p0 = jnp.max(p, axis=1, keepdims=True)
    i0 = jnp.min(jnp.where(p == p0, iota_e, _E), axis=1, keepdims=True)
    oh0 = iota_e == i0
    pm = jnp.where(oh0, -1.0, p)
    p1 = jnp.max(pm, axis=1, keepdims=True)
    i1 = jnp.min(jnp.where(pm == p1, iota_e, _E), axis=1, keepdims=True)
    oh1 = iota_e == i1
    s = p0 + p1 + 1e-8
    w0 = p0 / s
    w1 = p1 / s

    oh0f = oh0.astype(jnp.float32)
    oh1f = oh1.astype(jnp.float32)
    s2 = oh0f + oh1f
    row = lax.broadcasted_iota(jnp.int32, (_TB, _TB), 0)
    col = lax.broadcasted_iota(jnp.int32, (_TB, _TB), 1)
    stril = (row > col).astype(jnp.float32)
    c_in = jnp.dot(stril, s2, preferred_element_type=jnp.float32)
    carry = acc_ref[0:1, 0:_E]
    c_tot = c_in + carry
    pos0 = jnp.sum(c_tot * oh0f, axis=1, keepdims=True).astype(jnp.int32)
    pos1 = jnp.sum(c_tot * oh1f, axis=1, keepdims=True).astype(jnp.int32)
    acc_ref[0:1, 0:_E] = carry + jnp.sum(s2, axis=0, keepdims=True)
    acc_ref[1:2, 0:_E] += jnp.sum(p, axis=0, keepdims=True)

    q0 = jnp.where(pos0 < _CAP, i0 * _CAP + pos0, -1)
    q1 = jnp.where(pos1 < _CAP, i1 * _CAP + pos1, -1)
    q0_3 = jnp.expand_dims(q0, 2)
    q1_3 = jnp.expand_dims(q1, 2)
    w0_3 = jnp.expand_dims(w0, 2)
    w1_3 = jnp.expand_dims(w1, 2)
    ee = lax.broadcasted_iota(jnp.int32, (_TB, _E, _CAP), 1)
    cc = lax.broadcasted_iota(jnp.int32, (_TB, _E, _CAP), 2)
    qq = ee * _CAP + cc
    m0 = qq == q0_3
    m1 = qq == q1_3
    disp_ref[...] = m0.astype(jnp.float32) + m1.astype(jnp.float32)
    comb_ref[...] = jnp.where(m0, w0_3, 0.0) + jnp.where(m1, w1_3, 0.0)

    cnt = acc_ref[0:1, 0:_E]
    psum = acc_ref[1:2, 0:_E]
    aux_ref[...] = (0.1 * _E) * jnp.sum(
        (psum / _T) * (cnt / (_T * _TOPK)), axis=1, keepdims=True)


def kernel(hidden_states, W1, b1, W2, b2, expert_weights):
    Bv, Sv, Hv = hidden_states.shape
    x = hidden_states.reshape(Bv * Sv, Hv)

    disp, comb, probs, aux = pl.pallas_call(
        _router_body,
        grid=(_G,),
        in_specs=[
            pl.BlockSpec((_TB, _H), lambda i: (i, 0)),
            pl.BlockSpec((_H, _H), lambda i: (0, 0)),
            pl.BlockSpec((1, _H), lambda i: (0, 0)),
            pl.BlockSpec((_H, _E), lambda i: (0, 0)),
            pl.BlockSpec((1, _E), lambda i: (0, 0)),
            pl.BlockSpec((1, _E), lambda i: (0, 0)),
        ],
        out_specs=[
            pl.BlockSpec((_TB, _E, _CAP), lambda i: (i, 0, 0)),
            pl.BlockSpec((_TB, _E, _CAP), lambda i: (i, 0, 0)),
            pl.BlockSpec((_TB, _E), lambda i: (i, 0)),
            pl.BlockSpec((1, 1), lambda i: (0, 0)),
        ],
        out_shape=[
            jax.ShapeDtypeStruct((_T, _E, _CAP), jnp.float32),
            jax.ShapeDtypeStruct((_T, _E, _CAP), jnp.float32),
            jax.ShapeDtypeStruct((_T, _E), jnp.float32),
            jax.ShapeDtypeStruct((1, 1), jnp.float32),
        ],
        scratch_shapes=[pltpu.VMEM((8, 128), jnp.float32)],
    )(x, W1.T, b1.reshape(1, Hv), W2.T, b2.reshape(1, _E),
      expert_weights.reshape(1, _E))

    dispatch = disp.reshape(Bv, Sv, _E, _CAP)
    combine = comb.reshape(Bv, Sv, _E, _CAP)
    router_probs = probs.reshape(Bv, Sv, _E)
    return dispatch, combine, router_probs, aux.reshape(())


# speedup vs baseline: 1.6318x; 1.0523x over previous
import jax
import jax.numpy as jnp
from jax import lax
from jax.experimental import pallas as pl
from jax.experimental.pallas import tpu as pltpu

_H = 768
_E = 16
_TOPK = 2
_CAP = 256
_T = 2048
_TB = 512
_G = _T // _TB


def _router_body(x_ref, w1_ref, b1_ref, w2_ref, b2_ref, ew_ref,
                 disp_ref, comb_ref, probs_ref, aux_ref, acc_ref):
    i = pl.program_id(0)

    @pl.when(i == 0)
    def _init():
        acc_ref[...] = jnp.zeros_like(acc_ref)

    h = jnp.dot(x_ref[...], w1_ref[...], preferred_element_type=jnp.float32)
    h = jnp.maximum(h + b1_ref[...], 0.0)
    logits = jnp.dot(h, w2_ref[...], preferred_element_type=jnp.float32)
    logits = (logits + b2_ref[...]) * ew_ref[...]

    m = jnp.max(logits, axis=1, keepdims=True)
    ex = jnp.exp(logits - m)
    p = ex / jnp.sum(ex, axis=1, keepdims=True)
    probs_ref[...] = p

    iota_e = lax.broadcasted_iota(jnp.int32, (_TB, _E), 1)
    p0 = jnp.max(p, axis=1, keepdims=True)
    i0 = jnp.min(jnp.where(p == p0, iota_e, _E), axis=1, keepdims=True)
    oh0 = iota_e == i0
    pm = jnp.where(oh0, -1.0, p)
    p1 = jnp.max(pm, axis=1, keepdims=True)
    i1 = jnp.min(jnp.where(pm == p1, iota_e, _E), axis=1, keepdims=True)
    oh1 = iota_e == i1
    s = p0 + p1 + 1e-8
    w0 = p0 / s
    w1 = p1 / s

    oh0f = oh0.astype(jnp.float32)
    oh1f = oh1.astype(jnp.float32)
    s2 = oh0f + oh1f
    row = lax.broadcasted_iota(jnp.int32, (_TB, _TB), 0)
    col = lax.broadcasted_iota(jnp.int32, (_TB, _TB), 1)
    stril = (row > col).astype(jnp.float32)
    c_in = jnp.dot(stril, s2, preferred_element_type=jnp.float32)
    carry = acc_ref[0:1, 0:_E]
    c_tot = c_in + carry
    pos0 = jnp.sum(c_tot * oh0f, axis=1, keepdims=True).astype(jnp.int32)
    pos1 = jnp.sum(c_tot * oh1f, axis=1, keepdims=True).astype(jnp.int32)
    acc_ref[0:1, 0:_E] = carry + jnp.sum(s2, axis=0, keepdims=True)
    acc_ref[1:2, 0:_E] += jnp.sum(p, axis=0, keepdims=True)

    q0 = jnp.where(pos0 < _CAP, i0 * _CAP + pos0, -1)
    q1 = jnp.where(pos1 < _CAP, i1 * _CAP + pos1, -1)
    q0_3 = jnp.expand_dims(q0, 2)
    q1_3 = jnp.expand_dims(q1, 2)
    w0_3 = jnp.expand_dims(w0, 2)
    w1_3 = jnp.expand_dims(w1, 2)
    ee = lax.broadcasted_iota(jnp.int32, (_TB, _E, _CAP), 1)
    cc = lax.broadcasted_iota(jnp.int32, (_TB, _E, _CAP), 2)
    qq = ee * _CAP + cc
    m0 = qq == q0_3
    m1 = qq == q1_3
    disp_ref[...] = (m0 | m1).astype(jnp.float32)
    comb_ref[...] = jnp.where(m0, w0_3, jnp.where(m1, w1_3, 0.0))

    cnt = acc_ref[0:1, 0:_E]
    psum = acc_ref[1:2, 0:_E]
    aux_ref[...] = (0.1 * _E) * jnp.sum(
        (psum / _T) * (cnt / (_T * _TOPK)), axis=1, keepdims=True)


def kernel(hidden_states, W1, b1, W2, b2, expert_weights):
    Bv, Sv, Hv = hidden_states.shape
    x = hidden_states.reshape(Bv * Sv, Hv)

    disp, comb, probs, aux = pl.pallas_call(
        _router_body,
        grid=(_G,),
        in_specs=[
            pl.BlockSpec((_TB, _H), lambda i: (i, 0)),
            pl.BlockSpec((_H, _H), lambda i: (0, 0)),
            pl.BlockSpec((1, _H), lambda i: (0, 0)),
            pl.BlockSpec((_H, _E), lambda i: (0, 0)),
            pl.BlockSpec((1, _E), lambda i: (0, 0)),
            pl.BlockSpec((1, _E), lambda i: (0, 0)),
        ],
        out_specs=[
            pl.BlockSpec((_TB, _E, _CAP), lambda i: (i, 0, 0)),
            pl.BlockSpec((_TB, _E, _CAP), lambda i: (i, 0, 0)),
            pl.BlockSpec((_TB, _E), lambda i: (i, 0)),
            pl.BlockSpec((1, 1), lambda i: (0, 0)),
        ],
        out_shape=[
            jax.ShapeDtypeStruct((_T, _E, _CAP), jnp.float32),
            jax.ShapeDtypeStruct((_T, _E, _CAP), jnp.float32),
            jax.ShapeDtypeStruct((_T, _E), jnp.float32),
            jax.ShapeDtypeStruct((1, 1), jnp.float32),
        ],
        scratch_shapes=[pltpu.VMEM((8, 128), jnp.float32)],
    )(x, W1.T, b1.reshape(1, Hv), W2.T, b2.reshape(1, _E),
      expert_weights.reshape(1, _E))

    dispatch = disp.reshape(Bv, Sv, _E, _CAP)
    combine = comb.reshape(Bv, Sv, _E, _CAP)
    router_probs = probs.reshape(Bv, Sv, _E)
    return dispatch, combine, router_probs, aux.reshape(())
